# baseline (speedup 1.0000x reference)
import functools

import jax
import jax.numpy as jnp
from jax import lax
from jax.experimental import pallas as pl
from jax.experimental.pallas import tpu as pltpu
from jax.experimental.pallas import tpu_sc as plsc

N = 10000
E = 320000
D = 128
EPS = 1e-07

NC = 2
NS = 16
PER_TEC = E // NS
CHUNK = 80
NCHUNKS = PER_TEC // CHUNK
NROT = 2
NIDX = 4
UNROLL = 4
MAIN_ITERS = NCHUNKS // UNROLL
TAIL = NCHUNKS - MAIN_ITERS * UNROLL
ROWS_PER_TEC = 624
ROWS_REM = N - NS * ROWS_PER_TEC


def _prep_body(beta_ref, x_ref, t_ref):
    g = jnp.maximum(x_ref[...], 0.0) + EPS
    w = jnp.exp(beta_ref[0, 0] * g)
    t_ref[0] = w
    t_ref[1] = g * w


def _precompute_tables(x, beta):
    bn = 1000
    t = pl.pallas_call(
        _prep_body,
        grid=(N // bn,),
        in_specs=[
            pl.BlockSpec(memory_space=pltpu.SMEM),
            pl.BlockSpec((bn, D), lambda i: (i, 0)),
        ],
        out_specs=pl.BlockSpec((2, bn, D), lambda i: (0, i, 0)),
        out_shape=jax.ShapeDtypeStruct((2, N, D), jnp.float32),
    )(jnp.reshape(beta, (1, 1)).astype(jnp.float32), x)
    return t.reshape(2 * N, D)


def _sc_body(table, src4, dst4, out,
             sidx, didx, rows, acc, gsems, ssems, isems):
    c = lax.axis_index("c")
    s = lax.axis_index("s")
    w = c * NS + s

    def start_idx(j, q):
        pltpu.async_copy(src4.at[w * NCHUNKS + j], sidx.at[q], isems.at[q])
        pltpu.async_copy(dst4.at[s * NCHUNKS + j], didx.at[q], isems.at[q])

    def wait_idx(j, q):
        pltpu.make_async_copy(src4.at[w * NCHUNKS + j], sidx.at[q],
                              isems.at[q]).wait()
        pltpu.make_async_copy(dst4.at[s * NCHUNKS + j], didx.at[q],
                              isems.at[q]).wait()

    def start_gather(q, r):
        pltpu.async_copy(table.at[sidx.at[q]], rows.at[r], gsems.at[r])

    def wait_gather(q, r):
        pltpu.make_async_copy(table.at[sidx.at[q]], rows.at[r],
                              gsems.at[r]).wait()

    def start_scatter(q, r):
        pltpu.async_copy(rows.at[r], acc.at[didx.at[q]], ssems.at[r],
                         add=True)

    def wait_scatter(q, r):
        pltpu.make_async_copy(rows.at[r], acc.at[didx.at[q]],
                              ssems.at[r]).wait()

    for k in range(NROT):
        start_idx(k, k)

    zero16 = jnp.zeros((16,), jnp.float32)

    def zrow(i, carry):
        for kk in range(D // 16):
            rows[0, i, pl.ds(kk * 16, 16)] = zero16
        return carry

    lax.fori_loop(0, CHUNK, zrow, 0)
    r0 = s * ROWS_PER_TEC
    for off in range(0, ROWS_PER_TEC, CHUNK):
        sz = min(CHUNK, ROWS_PER_TEC - off)
        pltpu.sync_copy(rows.at[0, pl.ds(0, sz)],
                        acc.at[pl.ds(r0 + off, sz)])

    @pl.when(s == NS - 1)
    def _():
        pltpu.sync_copy(rows.at[0, pl.ds(0, ROWS_REM)],
                        acc.at[pl.ds(NS * ROWS_PER_TEC, ROWS_REM)])

    plsc.subcore_barrier()

    def body(j, k, first):
        r = k % NROT
        q = k % NIDX
        q3 = (k + NROT) % NIDX
        if not first:
            wait_scatter(q3, r)

            @pl.when(j + NROT < NCHUNKS)
            def _():
                start_idx(j + NROT, q3)
        else:
            start_idx(j + NROT, q3)
        wait_idx(j, q)
        start_gather(q, r)
        if not first or k > 0:
            rp = (k - 1) % NROT
            qp = (k - 1) % NIDX
            wait_gather(qp, rp)
            start_scatter(qp, rp)

    for k in range(UNROLL):
        body(k, k, first=(k < NROT))

    def loop_body(jj, carry):
        j0 = jj * UNROLL
        for k in range(UNROLL):
            body(j0 + k, k, first=False)
        return carry

    lax.fori_loop(1, MAIN_ITERS, loop_body, 0)

    for k in range(TAIL):
        body(MAIN_ITERS * UNROLL + k, k, first=False)

    last = NCHUNKS - 1
    wait_gather(last % NIDX, last % NROT)
    start_scatter(last % NIDX, last % NROT)
    for j in range(NCHUNKS - NROT, NCHUNKS):
        wait_scatter(j % NIDX, j % NROT)

    plsc.subcore_barrier()
    pltpu.sync_copy(acc.at[pl.ds(r0, ROWS_PER_TEC)],
                    out.at[pl.ds(c * N + r0, ROWS_PER_TEC)])

    @pl.when(s == NS - 1)
    def _():
        pltpu.sync_copy(acc.at[pl.ds(NS * ROWS_PER_TEC, ROWS_REM)],
                        out.at[pl.ds(c * N + NS * ROWS_PER_TEC, ROWS_REM)])


def _sc_aggregate(table, src, dst):
    src4 = jnp.concatenate([src, src + N]).reshape(NC * NS * NCHUNKS, CHUNK)
    dst4 = dst.reshape(NS * NCHUNKS, CHUNK)
    mesh = plsc.VectorSubcoreMesh(core_axis_name="c", subcore_axis_name="s")
    run = functools.partial(
        pl.kernel,
        out_type=jax.ShapeDtypeStruct((2 * N, D), jnp.float32),
        mesh=mesh,
        scratch_types=[
            pltpu.VMEM((NIDX, CHUNK), jnp.int32),
            pltpu.VMEM((NIDX, CHUNK), jnp.int32),
            pltpu.VMEM((NROT, CHUNK, D), jnp.float32),
            pltpu.VMEM_SHARED((N, D), jnp.float32),
            pltpu.SemaphoreType.DMA((NROT,)),
            pltpu.SemaphoreType.DMA((NROT,)),
            pltpu.SemaphoreType.DMA((NIDX,)),
        ],
    )(_sc_body)
    return run(table, src4, dst4)


def _finish_body(s1_ref, s2_ref, w_ref, b_ref, o_ref):
    s1 = s1_ref[...]
    s2 = s2_ref[...]
    agg = jnp.where(s1 > 0.0, s2 / jnp.where(s1 > 0.0, s1, 1.0), 0.0)
    acc = lax.dot_general(agg, w_ref[...], (((1,), (1,)), ((), ())),
                          preferred_element_type=jnp.float32)
    o_ref[...] = acc + b_ref[...]


def _finish(acc, W, b):
    bn = 1000
    return pl.pallas_call(
        _finish_body,
        grid=(N // bn,),
        in_specs=[
            pl.BlockSpec((bn, D), lambda i: (i, 0)),
            pl.BlockSpec((bn, D), lambda i: (i + N // bn, 0)),
            pl.BlockSpec((D, D), lambda i: (0, 0)),
            pl.BlockSpec((1, D), lambda i: (0, 0)),
        ],
        out_specs=pl.BlockSpec((bn, D), lambda i: (i, 0)),
        out_shape=jax.ShapeDtypeStruct((N, D), jnp.float32),
    )(acc, acc, W, jnp.reshape(b, (1, D)))


def kernel(x, edge_index, W, b, beta):
    dst = edge_index[0].astype(jnp.int32)
    src = edge_index[1].astype(jnp.int32)
    table = _precompute_tables(x, beta)
    acc = _sc_aggregate(table, src, dst)
    return _finish(acc, W, b)


# speedup vs baseline: 1.1613x; 1.1613x over previous
import functools

import jax
import jax.numpy as jnp
from jax import lax
from jax.experimental import pallas as pl
from jax.experimental.pallas import tpu as pltpu
from jax.experimental.pallas import tpu_sc as plsc

N = 10000
E = 320000
D = 128
EPS = 1e-07

NC = 2
NS = 16
PER_TEC = E // NS
CHUNK = 80
NCHUNKS = PER_TEC // CHUNK
NROT = 3
NIDX = 6
UNROLL = 6
MAIN_ITERS = NCHUNKS // UNROLL
TAIL = NCHUNKS - MAIN_ITERS * UNROLL
ROWS_PER_TEC = 624
ROWS_REM = N - NS * ROWS_PER_TEC


def _prep_body(beta_ref, x_ref, t_ref):
    g = jnp.maximum(x_ref[...], 0.0) + EPS
    w = jnp.exp(beta_ref[0, 0] * g)
    t_ref[0] = w
    t_ref[1] = g * w


def _precompute_tables(x, beta):
    bn = 1000
    t = pl.pallas_call(
        _prep_body,
        grid=(N // bn,),
        in_specs=[
            pl.BlockSpec(memory_space=pltpu.SMEM),
            pl.BlockSpec((bn, D), lambda i: (i, 0)),
        ],
        out_specs=pl.BlockSpec((2, bn, D), lambda i: (0, i, 0)),
        out_shape=jax.ShapeDtypeStruct((2, N, D), jnp.float32),
    )(jnp.reshape(beta, (1, 1)).astype(jnp.float32), x)
    return t.reshape(2 * N, D)


def _sc_body(table, src4, dst4, out,
             sidx, didx, rows, acc, gsems, ssems, isems):
    c = lax.axis_index("c")
    s = lax.axis_index("s")
    w = c * NS + s

    def start_idx(j, q):
        pltpu.async_copy(src4.at[w * NCHUNKS + j], sidx.at[q], isems.at[q])
        pltpu.async_copy(dst4.at[s * NCHUNKS + j], didx.at[q], isems.at[q])

    def wait_idx(j, q):
        pltpu.make_async_copy(src4.at[w * NCHUNKS + j], sidx.at[q],
                              isems.at[q]).wait()
        pltpu.make_async_copy(dst4.at[s * NCHUNKS + j], didx.at[q],
                              isems.at[q]).wait()

    def start_gather(q, r):
        pltpu.async_copy(table.at[sidx.at[q]], rows.at[r], gsems.at[r])

    def wait_gather(q, r):
        pltpu.make_async_copy(table.at[sidx.at[q]], rows.at[r],
                              gsems.at[r]).wait()

    def start_scatter(q, r):
        pltpu.async_copy(rows.at[r], acc.at[didx.at[q]], ssems.at[r],
                         add=True)

    def wait_scatter(q, r):
        pltpu.make_async_copy(rows.at[r], acc.at[didx.at[q]],
                              ssems.at[r]).wait()

    for k in range(NROT):
        start_idx(k, k)

    zero16 = jnp.zeros((16,), jnp.float32)

    def zrow(i, carry):
        for kk in range(D // 16):
            rows[0, i, pl.ds(kk * 16, 16)] = zero16
        return carry

    lax.fori_loop(0, CHUNK, zrow, 0)
    r0 = s * ROWS_PER_TEC
    for off in range(0, ROWS_PER_TEC, CHUNK):
        sz = min(CHUNK, ROWS_PER_TEC - off)
        pltpu.sync_copy(rows.at[0, pl.ds(0, sz)],
                        acc.at[pl.ds(r0 + off, sz)])

    @pl.when(s == NS - 1)
    def _():
        pltpu.sync_copy(rows.at[0, pl.ds(0, ROWS_REM)],
                        acc.at[pl.ds(NS * ROWS_PER_TEC, ROWS_REM)])

    plsc.subcore_barrier()

    def body(j, k, first):
        r = k % NROT
        q = k % NIDX
        q3 = (k + NROT) % NIDX
        if not first:
            wait_scatter(q3, r)

            @pl.when(j + NROT < NCHUNKS)
            def _():
                start_idx(j + NROT, q3)
        else:
            start_idx(j + NROT, q3)
        wait_idx(j, q)
        start_gather(q, r)
        if not first or k > 0:
            rp = (k - 1) % NROT
            qp = (k - 1) % NIDX
            wait_gather(qp, rp)
            start_scatter(qp, rp)

    for k in range(UNROLL):
        body(k, k, first=(k < NROT))

    def loop_body(jj, carry):
        j0 = jj * UNROLL
        for k in range(UNROLL):
            body(j0 + k, k, first=False)
        return carry

    lax.fori_loop(1, MAIN_ITERS, loop_body, 0)

    for k in range(TAIL):
        body(MAIN_ITERS * UNROLL + k, k, first=False)

    last = NCHUNKS - 1
    wait_gather(last % NIDX, last % NROT)
    start_scatter(last % NIDX, last % NROT)
    for j in range(NCHUNKS - NROT, NCHUNKS):
        wait_scatter(j % NIDX, j % NROT)

    plsc.subcore_barrier()
    pltpu.sync_copy(acc.at[pl.ds(r0, ROWS_PER_TEC)],
                    out.at[pl.ds(c * N + r0, ROWS_PER_TEC)])

    @pl.when(s == NS - 1)
    def _():
        pltpu.sync_copy(acc.at[pl.ds(NS * ROWS_PER_TEC, ROWS_REM)],
                        out.at[pl.ds(c * N + NS * ROWS_PER_TEC, ROWS_REM)])


def _sc_aggregate(table, src, dst):
    src4 = jnp.concatenate([src, src + N]).reshape(NC * NS * NCHUNKS, CHUNK)
    dst4 = dst.reshape(NS * NCHUNKS, CHUNK)
    mesh = plsc.VectorSubcoreMesh(core_axis_name="c", subcore_axis_name="s")
    run = functools.partial(
        pl.kernel,
        out_type=jax.ShapeDtypeStruct((2 * N, D), jnp.float32),
        mesh=mesh,
        scratch_types=[
            pltpu.VMEM((NIDX, CHUNK), jnp.int32),
            pltpu.VMEM((NIDX, CHUNK), jnp.int32),
            pltpu.VMEM((NROT, CHUNK, D), jnp.float32),
            pltpu.VMEM_SHARED((N, D), jnp.float32),
            pltpu.SemaphoreType.DMA((NROT,)),
            pltpu.SemaphoreType.DMA((NROT,)),
            pltpu.SemaphoreType.DMA((NIDX,)),
        ],
    )(_sc_body)
    return run(table, src4, dst4)


def _finish_body(s1_ref, s2_ref, w_ref, b_ref, o_ref):
    s1 = s1_ref[...]
    s2 = s2_ref[...]
    agg = jnp.where(s1 > 0.0, s2 / jnp.where(s1 > 0.0, s1, 1.0), 0.0)
    acc = lax.dot_general(agg, w_ref[...], (((1,), (1,)), ((), ())),
                          preferred_element_type=jnp.float32)
    o_ref[...] = acc + b_ref[...]


def _finish(acc, W, b):
    bn = 1000
    return pl.pallas_call(
        _finish_body,
        grid=(N // bn,),
        in_specs=[
            pl.BlockSpec((bn, D), lambda i: (i, 0)),
            pl.BlockSpec((bn, D), lambda i: (i + N // bn, 0)),
            pl.BlockSpec((D, D), lambda i: (0, 0)),
            pl.BlockSpec((1, D), lambda i: (0, 0)),
        ],
        out_specs=pl.BlockSpec((bn, D), lambda i: (i, 0)),
        out_shape=jax.ShapeDtypeStruct((N, D), jnp.float32),
    )(acc, acc, W, jnp.reshape(b, (1, D)))


def kernel(x, edge_index, W, b, beta):
    dst = edge_index[0].astype(jnp.int32)
    src = edge_index[1].astype(jnp.int32)
    table = _precompute_tables(x, beta)
    acc = _sc_aggregate(table, src, dst)
    return _finish(acc, W, b)
